# baseline (speedup 1.0000x reference)
import functools

import jax
import jax.numpy as jnp
from jax import lax
from jax.experimental import pallas as pl
from jax.experimental.pallas import tpu as pltpu
from jax.experimental.pallas import tpu_sc as plsc

_B = 16384
_NNUM = 13
_NCAT = 26
_D = 64
_ROWS = _NNUM + _NCAT
_L = 16
_NC = 2
_NS = 16
_NW = _NC * _NS
_PB = _B // _NW
_CB = 16
_NCH = _PB // _CB
_DG = _D // 8
_BT = _B // 128

_mesh = plsc.VectorSubcoreMesh(
    core_axis_name="c", subcore_axis_name="s", num_cores=_NC, num_subcores=_NS
)


@functools.partial(
    pl.kernel,
    out_type=jax.ShapeDtypeStruct((_ROWS, _DG, _BT, 8, 128), jnp.float32),
    mesh=_mesh,
    scratch_types=[
        pltpu.VMEM((_NCAT, _PB), jnp.int32),
        pltpu.VMEM((_NNUM, _PB), jnp.float32),
        pltpu.VMEM((_NNUM, _D), jnp.float32),
        pltpu.VMEM((_NNUM, _D), jnp.float32),
        pltpu.VMEM((_NNUM, _D, _L), jnp.float32),
        pltpu.VMEM((_NNUM, _D, _L), jnp.float32),
        pltpu.VMEM((_NCAT * _CB, _D), jnp.float32),
        pltpu.VMEM((_ROWS, _DG, 8, _CB), jnp.float32),
        pltpu.SemaphoreType.DMA,
        pltpu.SemaphoreType.DMA,
        pltpu.SemaphoreType.DMA,
    ],
    compiler_params=pltpu.CompilerParams(
        use_tc_tiling_on_sc=False, needs_layout_passes=False
    ),
)
def _tokenizer_sc(table_hbm, idxT_hbm, xt_hbm, w_hbm, bias_hbm, out_hbm,
                  idxT_v, xt_v, w_v, b_v, wbc_v, bbc_v, gbuf, obuf,
                  gsem, osem, ssem):
    wid = lax.axis_index("s") * _NC + lax.axis_index("c")
    base = wid * _PB

    pltpu.async_copy(idxT_hbm.at[:, pl.ds(base, _PB)], idxT_v, ssem)
    pltpu.async_copy(xt_hbm.at[:, pl.ds(base, _PB)], xt_v, ssem)
    pltpu.async_copy(w_hbm, w_v, ssem)
    pltpu.async_copy(bias_hbm, b_v, ssem).wait()
    pltpu.make_async_copy(idxT_hbm.at[:, pl.ds(base, _PB)], idxT_v, ssem).wait()
    pltpu.make_async_copy(xt_hbm.at[:, pl.ds(base, _PB)], xt_v, ssem).wait()
    pltpu.make_async_copy(w_hbm, w_v, ssem).wait()

    iota = lax.iota(jnp.int32, _L)

    for i in range(_NNUM):
        for j in range(_D // _L):
            wv = w_v[i, pl.ds(j * _L, _L)]
            bv = b_v[i, pl.ds(j * _L, _L)]
            for k in range(_L):
                wbc_v[i, j * _L + k, :] = jnp.full((_L,), wv[k], jnp.float32)
                bbc_v[i, j * _L + k, :] = jnp.full((_L,), bv[k], jnp.float32)

    def fire_gathers(row0):
        for c in range(_NCAT):
            pltpu.async_copy(
                table_hbm.at[idxT_v.at[c, pl.ds(row0, _CB)]],
                gbuf.at[pl.ds(c * _CB, _CB)],
                gsem,
            )

    fire_gathers(0)

    def out_slice(bt, bc0):
        return out_hbm.at[:, :, bt, :, pl.ds(bc0, _CB)]

    @pl.loop(0, _NCH)
    def _chunk(s):
        row0 = s * _CB
        bt = wid * (_PB // 128) + s // (128 // _CB)
        bc0 = (s % (128 // _CB)) * _CB

        @pl.when(s > 0)
        def _():
            pltpu.make_async_copy(obuf, out_slice(bt, bc0), osem).wait()

        for i in range(_NNUM):
            xv = xt_v[i, pl.ds(row0, _CB)]
            for d in range(_D):
                obuf[i, d // 8, d % 8, :] = (
                    xv * wbc_v[i, d, :] + bbc_v[i, d, :]
                )

        pltpu.make_async_copy(
            table_hbm.at[pl.ds(0, _NCAT * _CB)], gbuf, gsem
        ).wait()

        @pl.loop(0, _NCAT)
        def _block(c):
            rowv = c * _CB + iota
            colv = jnp.zeros((_L,), jnp.int32)
            for d in range(_D):
                vec = plsc.load_gather(gbuf, [rowv, colv])
                obuf[_NNUM + c, d // 8, d % 8, :] = vec
                if d != _D - 1:
                    colv = colv + 1

            @pl.when(s < _NCH - 1)
            def _():
                pltpu.async_copy(
                    table_hbm.at[idxT_v.at[c, pl.ds(row0 + _CB, _CB)]],
                    gbuf.at[pl.ds(c * _CB, _CB)],
                    gsem,
                )

        pltpu.async_copy(obuf, out_slice(bt, bc0), osem)

    pltpu.make_async_copy(
        obuf, out_hbm.at[:, :, 0, :, pl.ds(0, _CB)], osem
    ).wait()


def kernel(x_num, x_cat, W_num, b_num, cat_table):
    idxT = x_cat.T.astype(jnp.int32)
    xt = x_num[:, :, 0].T
    out_phys = _tokenizer_sc(cat_table, idxT, xt, W_num, b_num)
    return out_phys.transpose(2, 4, 0, 1, 3).reshape(_B, _ROWS, _D)


# speedup vs baseline: 8.7254x; 8.7254x over previous
import functools

import jax
import jax.numpy as jnp
from jax import lax
from jax.experimental import pallas as pl
from jax.experimental.pallas import tpu as pltpu
from jax.experimental.pallas import tpu_sc as plsc

_B = 16384
_NNUM = 13
_NCAT = 26
_D = 64
_ROWS = _NNUM + _NCAT
_L = 16
_NC = 2
_NS = 16
_NW = _NC * _NS
_PB = _B // _NW
_CB = 16
_NCH = _PB // _CB
_DG = _D // 8
_BT = _B // 128

_mesh = plsc.VectorSubcoreMesh(
    core_axis_name="c", subcore_axis_name="s", num_cores=_NC, num_subcores=_NS
)


@functools.partial(
    pl.kernel,
    out_type=jax.ShapeDtypeStruct((_ROWS, _DG, _BT, 8, 128), jnp.float32),
    mesh=_mesh,
    scratch_types=[
        pltpu.VMEM((_NCAT, _PB), jnp.int32),
        pltpu.VMEM((_NNUM, _PB), jnp.float32),
        pltpu.VMEM((_NNUM, _D), jnp.float32),
        pltpu.VMEM((_NNUM, _D), jnp.float32),
        pltpu.VMEM((_NNUM, _D, _L), jnp.float32),
        pltpu.VMEM((_NNUM, _D, _L), jnp.float32),
        pltpu.VMEM((_NCAT * _CB, _D), jnp.float32),
        pltpu.VMEM((_NNUM, _DG, 8, _CB), jnp.float32),
        pltpu.VMEM((_NCAT, _DG, 8, _CB), jnp.float32),
        pltpu.SemaphoreType.DMA,
        pltpu.SemaphoreType.DMA,
        pltpu.SemaphoreType.DMA,
        pltpu.SemaphoreType.DMA,
    ],
    compiler_params=pltpu.CompilerParams(
        use_tc_tiling_on_sc=False, needs_layout_passes=False
    ),
)
def _tokenizer_sc(table_hbm, idxT_hbm, xt_hbm, w_hbm, bias_hbm, out_hbm,
                  idxT_v, xt_v, w_v, b_v, wbc_v, bbc_v, gbuf, nbuf, cbuf,
                  gsem, nsem, csem, ssem):
    wid = lax.axis_index("s") * _NC + lax.axis_index("c")
    base = wid * _PB

    pltpu.async_copy(idxT_hbm.at[:, pl.ds(base, _PB)], idxT_v, ssem)
    pltpu.async_copy(xt_hbm.at[:, pl.ds(base, _PB)], xt_v, ssem)
    pltpu.async_copy(w_hbm, w_v, ssem)
    pltpu.async_copy(bias_hbm, b_v, ssem).wait()
    pltpu.make_async_copy(idxT_hbm.at[:, pl.ds(base, _PB)], idxT_v, ssem).wait()
    pltpu.make_async_copy(xt_hbm.at[:, pl.ds(base, _PB)], xt_v, ssem).wait()
    pltpu.make_async_copy(w_hbm, w_v, ssem).wait()

    iota = lax.iota(jnp.int32, _L)

    for i in range(_NNUM):
        for j in range(_D // _L):
            wv = w_v[i, pl.ds(j * _L, _L)]
            bv = b_v[i, pl.ds(j * _L, _L)]
            for k in range(_L):
                wbc_v[i, j * _L + k, :] = jnp.full((_L,), wv[k], jnp.float32)
                bbc_v[i, j * _L + k, :] = jnp.full((_L,), bv[k], jnp.float32)

    def fire_gathers(row0):
        for c in range(_NCAT):
            pltpu.async_copy(
                table_hbm.at[idxT_v.at[c, pl.ds(row0, _CB)]],
                gbuf.at[pl.ds(c * _CB, _CB)],
                gsem,
            )

    fire_gathers(0)

    def num_slice(bt, bc0):
        return out_hbm.at[pl.ds(0, _NNUM), :, bt, :, pl.ds(bc0, _CB)]

    def cat_slice(bt, bc0):
        return out_hbm.at[pl.ds(_NNUM, _NCAT), :, bt, :, pl.ds(bc0, _CB)]

    @pl.loop(0, _NCH)
    def _chunk(s):
        row0 = s * _CB
        bt = wid * (_PB // 128) + s // (128 // _CB)
        bc0 = (s % (128 // _CB)) * _CB

        @pl.when(s > 0)
        def _():
            pltpu.make_async_copy(nbuf, num_slice(bt, bc0), nsem).wait()

        for i in range(_NNUM):
            xv = xt_v[i, pl.ds(row0, _CB)]
            for d in range(_D):
                nbuf[i, d // 8, d % 8, :] = xv * wbc_v[i, d, :] + bbc_v[i, d, :]

        pltpu.async_copy(nbuf, num_slice(bt, bc0), nsem)

        pltpu.make_async_copy(
            table_hbm.at[pl.ds(0, _NCAT * _CB)], gbuf, gsem
        ).wait()

        @pl.when(s > 0)
        def _():
            pltpu.make_async_copy(cbuf, cat_slice(bt, bc0), csem).wait()

        @pl.loop(0, _NCAT)
        def _block(c):
            rowv = c * _CB + iota
            colvs = [jnp.full((_L,), q, jnp.int32) for q in range(4)]
            for dq in range(_D // 4):
                vecs = [plsc.load_gather(gbuf, [rowv, colvs[q]])
                        for q in range(4)]
                for q in range(4):
                    d = 4 * dq + q
                    cbuf[c, d // 8, d % 8, :] = vecs[q]
                if dq != _D // 4 - 1:
                    colvs = [cv + 4 for cv in colvs]

            @pl.when(s < _NCH - 1)
            def _():
                pltpu.async_copy(
                    table_hbm.at[idxT_v.at[c, pl.ds(row0 + _CB, _CB)]],
                    gbuf.at[pl.ds(c * _CB, _CB)],
                    gsem,
                )

        pltpu.async_copy(cbuf, cat_slice(bt, bc0), csem)

    pltpu.make_async_copy(nbuf, num_slice(0, 0), nsem).wait()
    pltpu.make_async_copy(cbuf, cat_slice(0, 0), csem).wait()


def kernel(x_num, x_cat, W_num, b_num, cat_table):
    idxT = x_cat.T.astype(jnp.int32)
    xt = x_num[:, :, 0].T
    out_phys = _tokenizer_sc(cat_table, idxT, xt, W_num, b_num)
    return out_phys.transpose(2, 4, 0, 1, 3).reshape(_B, _ROWS, _D)
